# Initial kernel scaffold; baseline (speedup 1.0000x reference)
import jax
import jax.numpy as jnp
from jax.experimental import pallas as pl


def kernel(channel_indices, weight):
    raise NotImplementedError("write your pallas kernel here")


# speedup vs baseline: 3.3553x; 3.3553x over previous
import functools

import jax
import jax.numpy as jnp
from jax import lax
from jax.experimental import pallas as pl
from jax.experimental.pallas import tpu as pltpu
from jax.experimental.pallas import tpu_sc as plsc

N_ROWS = 16384 * 100
V, D = 256, 64
NC, NS, L = 2, 16, 16
NW = NC * NS
ROWS_PER_W = N_ROWS // NW
CH = 512
NCHUNKS = ROWS_PER_W // CH
GROUPS = CH // L

_mesh = plsc.VectorSubcoreMesh(
    core_axis_name="c", subcore_axis_name="s", num_cores=NC, num_subcores=NS
)


@functools.partial(
    pl.kernel,
    out_type=jax.ShapeDtypeStruct((N_ROWS * D,), jnp.float32),
    mesh=_mesh,
    compiler_params=pltpu.CompilerParams(needs_layout_passes=False),
    scratch_types=[
        pltpu.VMEM((V * D,), jnp.float32),
        pltpu.VMEM((CH,), jnp.int32),
        pltpu.VMEM((CH,), jnp.int32),
        pltpu.VMEM((CH * D,), jnp.float32),
        pltpu.VMEM((CH * D,), jnp.float32),
        pltpu.SemaphoreType.DMA,
        pltpu.SemaphoreType.DMA,
        pltpu.SemaphoreType.DMA,
        pltpu.SemaphoreType.DMA,
    ],
)
def _sc_embed(w_hbm, idx_hbm, out_hbm, table_v, idx0, idx1, ob0, ob1,
              isem0, isem1, osem0, osem1):
    wid = lax.axis_index("s") * NC + lax.axis_index("c")
    base = wid * ROWS_PER_W

    pltpu.sync_copy(w_hbm, table_v)

    idx_bufs = (idx0, idx1)
    out_bufs = (ob0, ob1)
    isems = (isem0, isem1)
    osems = (osem0, osem1)

    iota16 = lax.iota(jnp.int32, L)
    obase0 = iota16 * D

    for b in range(2):
        pltpu.async_copy(
            idx_hbm.at[pl.ds(base + b * CH, CH)], idx_bufs[b], isems[b]
        )

    def compute_chunk(idx_buf, out_buf):
        def group(g, carry):
            rowvec = idx_buf[pl.ds(g * L, L)]
            tbase = rowvec * D
            obase = obase0 + g * (L * D)
            for k in range(D):
                vals = plsc.load_gather(table_v, [tbase + k])
                plsc.store_scatter(out_buf, [obase + k], vals)
            return carry

        lax.fori_loop(0, GROUPS, group, 0)

    def super_step(s, carry):
        for b in range(2):
            c = 2 * s + b
            pltpu.make_async_copy(
                idx_hbm.at[pl.ds(0, CH)], idx_bufs[b], isems[b]
            ).wait()

            @pl.when(s > 0)
            def _drain():
                pltpu.make_async_copy(
                    out_bufs[b], out_hbm.at[pl.ds(0, CH * D)], osems[b]
                ).wait()

            compute_chunk(idx_bufs[b], out_bufs[b])

            pltpu.async_copy(
                out_bufs[b],
                out_hbm.at[pl.ds((base + c * CH) * D, CH * D)],
                osems[b],
            )

            @pl.when(c + 2 < NCHUNKS)
            def _prefetch():
                pltpu.async_copy(
                    idx_hbm.at[pl.ds(base + (c + 2) * CH, CH)],
                    idx_bufs[b],
                    isems[b],
                )
        return carry

    lax.fori_loop(0, NCHUNKS // 2, super_step, 0)

    for b in range(2):
        pltpu.make_async_copy(
            out_bufs[b], out_hbm.at[pl.ds(0, CH * D)], osems[b]
        ).wait()


def kernel(channel_indices, weight):
    idx = channel_indices.reshape(-1).astype(jnp.int32)
    w = weight.reshape(-1)
    out = _sc_embed(w, idx)
    return out.reshape(channel_indices.shape + (weight.shape[1],))


# speedup vs baseline: 3.7461x; 1.1165x over previous
import functools

import jax
import jax.numpy as jnp
from jax import lax
from jax.experimental import pallas as pl
from jax.experimental.pallas import tpu as pltpu
from jax.experimental.pallas import tpu_sc as plsc

N_ROWS = 16384 * 100
V, D = 256, 64
NC, NS, L = 2, 16, 16
NW = NC * NS
ROWS_PER_W = N_ROWS // NW
CH = 512
NCHUNKS = ROWS_PER_W // CH
GROUPS = CH // L

_mesh = plsc.VectorSubcoreMesh(
    core_axis_name="c", subcore_axis_name="s", num_cores=NC, num_subcores=NS
)


@functools.partial(
    pl.kernel,
    out_type=jax.ShapeDtypeStruct((N_ROWS * D,), jnp.float32),
    mesh=_mesh,
    compiler_params=pltpu.CompilerParams(needs_layout_passes=False),
    scratch_types=[
        pltpu.VMEM((V * D,), jnp.float32),
        pltpu.VMEM((CH,), jnp.int32),
        pltpu.VMEM((CH,), jnp.int32),
        pltpu.VMEM((CH * D,), jnp.float32),
        pltpu.VMEM((CH * D,), jnp.float32),
        pltpu.SemaphoreType.DMA,
        pltpu.SemaphoreType.DMA,
        pltpu.SemaphoreType.DMA,
        pltpu.SemaphoreType.DMA,
    ],
)
def _sc_embed(w_hbm, idx_hbm, out_hbm, table_v, idx0, idx1, ob0, ob1,
              isem0, isem1, osem0, osem1):
    wid = lax.axis_index("s") * NC + lax.axis_index("c")
    base = wid * ROWS_PER_W

    pltpu.sync_copy(w_hbm, table_v)

    idx_bufs = (idx0, idx1)
    out_bufs = (ob0, ob1)
    isems = (isem0, isem1)
    osems = (osem0, osem1)

    iota16 = lax.iota(jnp.int32, L)
    obase0 = iota16 * D

    for b in range(2):
        pltpu.async_copy(
            idx_hbm.at[pl.ds(base + b * CH, CH)], idx_bufs[b], isems[b]
        )

    def compute_chunk(idx_buf, out_buf):
        @plsc.parallel_loop(0, GROUPS, unroll=2)
        def group(g):
            rowvec = idx_buf[pl.ds(g * L, L)]
            tbase = rowvec * D
            obase = obase0 + g * (L * D)
            for kb in range(0, D, 16):
                vals = [
                    plsc.load_gather(table_v, [tbase + (kb + t)])
                    for t in range(16)
                ]
                for t in range(16):
                    plsc.store_scatter(out_buf, [obase + (kb + t)], vals[t])

    def super_step(s, carry):
        for b in range(2):
            c = 2 * s + b
            pltpu.make_async_copy(
                idx_hbm.at[pl.ds(0, CH)], idx_bufs[b], isems[b]
            ).wait()

            @pl.when(s > 0)
            def _drain():
                pltpu.make_async_copy(
                    out_bufs[b], out_hbm.at[pl.ds(0, CH * D)], osems[b]
                ).wait()

            compute_chunk(idx_bufs[b], out_bufs[b])

            pltpu.async_copy(
                out_bufs[b],
                out_hbm.at[pl.ds((base + c * CH) * D, CH * D)],
                osems[b],
            )

            @pl.when(c + 2 < NCHUNKS)
            def _prefetch():
                pltpu.async_copy(
                    idx_hbm.at[pl.ds(base + (c + 2) * CH, CH)],
                    idx_bufs[b],
                    isems[b],
                )
        return carry

    lax.fori_loop(0, NCHUNKS // 2, super_step, 0)

    for b in range(2):
        pltpu.make_async_copy(
            out_bufs[b], out_hbm.at[pl.ds(0, CH * D)], osems[b]
        ).wait()


def kernel(channel_indices, weight):
    idx = channel_indices.reshape(-1).astype(jnp.int32)
    w = weight.reshape(-1)
    out = _sc_embed(w, idx)
    return out.reshape(channel_indices.shape + (weight.shape[1],))


# speedup vs baseline: 12.9976x; 3.4697x over previous
import functools

import jax
import jax.numpy as jnp
from jax import lax
from jax.experimental import pallas as pl
from jax.experimental.pallas import tpu as pltpu
from jax.experimental.pallas import tpu_sc as plsc

N_ROWS = 16384 * 100
V, D = 256, 64
NC, NS, L = 2, 16, 16
NW = NC * NS
ROWS_PER_W = N_ROWS // NW
CH = 512
NCHUNKS = ROWS_PER_W // CH
GROUPS = CH // L

_mesh = plsc.VectorSubcoreMesh(
    core_axis_name="c", subcore_axis_name="s", num_cores=NC, num_subcores=NS
)


@functools.partial(
    pl.kernel,
    out_type=jax.ShapeDtypeStruct((N_ROWS * D,), jnp.float32),
    mesh=_mesh,
    compiler_params=pltpu.CompilerParams(needs_layout_passes=False),
    scratch_types=[
        pltpu.VMEM((V * D,), jnp.float32),
        pltpu.VMEM((CH,), jnp.int32),
        pltpu.VMEM((CH,), jnp.int32),
        pltpu.VMEM((CH * D,), jnp.float32),
        pltpu.VMEM((CH * D,), jnp.float32),
        pltpu.SemaphoreType.DMA,
        pltpu.SemaphoreType.DMA,
        pltpu.SemaphoreType.DMA,
        pltpu.SemaphoreType.DMA,
    ],
)
def _sc_embed(w_hbm, idx_hbm, out_hbm, table_v, idx0, idx1, ob0, ob1,
              isem0, isem1, osem0, osem1):
    wid = lax.axis_index("s") * NC + lax.axis_index("c")
    base = wid * ROWS_PER_W

    pltpu.sync_copy(w_hbm, table_v)

    idx_bufs = (idx0, idx1)
    out_bufs = (ob0, ob1)
    isems = (isem0, isem1)
    osems = (osem0, osem1)

    iota16 = lax.iota(jnp.int32, L)
    obase0 = iota16 * D

    for b in range(2):
        pltpu.async_copy(
            idx_hbm.at[pl.ds(base + b * CH, CH)], idx_bufs[b], isems[b]
        )

    def compute_chunk(idx_buf, out_buf):
        @plsc.parallel_loop(0, GROUPS, unroll=2)
        def group(g):
            rowvec = idx_buf[pl.ds(g * L, L)] * D
            obase = g * (L * D)
            for l in range(L):
                tbase = rowvec[l]
                for j in range(0, D, L):
                    out_buf[pl.ds(obase + l * D + j, L)] = table_v[
                        pl.ds(tbase + j, L)
                    ]

    def super_step(s, carry):
        for b in range(2):
            c = 2 * s + b
            pltpu.make_async_copy(
                idx_hbm.at[pl.ds(0, CH)], idx_bufs[b], isems[b]
            ).wait()

            @pl.when(s > 0)
            def _drain():
                pltpu.make_async_copy(
                    out_bufs[b], out_hbm.at[pl.ds(0, CH * D)], osems[b]
                ).wait()

            compute_chunk(idx_bufs[b], out_bufs[b])

            pltpu.async_copy(
                out_bufs[b],
                out_hbm.at[pl.ds((base + c * CH) * D, CH * D)],
                osems[b],
            )

            @pl.when(c + 2 < NCHUNKS)
            def _prefetch():
                pltpu.async_copy(
                    idx_hbm.at[pl.ds(base + (c + 2) * CH, CH)],
                    idx_bufs[b],
                    isems[b],
                )
        return carry

    lax.fori_loop(0, NCHUNKS // 2, super_step, 0)

    for b in range(2):
        pltpu.make_async_copy(
            out_bufs[b], out_hbm.at[pl.ds(0, CH * D)], osems[b]
        ).wait()


def kernel(channel_indices, weight):
    idx = channel_indices.reshape(-1).astype(jnp.int32)
    w = weight.reshape(-1)
    out = _sc_embed(w, idx)
    return out.reshape(channel_indices.shape + (weight.shape[1],))


# speedup vs baseline: 31.1745x; 2.3985x over previous
import functools

import jax
import jax.numpy as jnp
from jax import lax
from jax.experimental import pallas as pl
from jax.experimental.pallas import tpu as pltpu
from jax.experimental.pallas import tpu_sc as plsc

NI, NJ = 16384, 100
V, D = 256, 64
NC, NS, L = 2, 16, 16
NW = NC * NS
IPW = NI // NW
NIB = IPW // 128
NKB = D // 8
NTILES = NKB * NIB
NSEG = IPW // L

_mesh = plsc.VectorSubcoreMesh(
    core_axis_name="c", subcore_axis_name="s", num_cores=NC, num_subcores=NS
)


@functools.partial(
    pl.kernel,
    out_type=jax.ShapeDtypeStruct((NJ, D, NI), jnp.float32),
    mesh=_mesh,
    compiler_params=pltpu.CompilerParams(needs_layout_passes=False),
    scratch_types=[
        pltpu.VMEM((D * V,), jnp.float32),
        pltpu.VMEM((IPW,), jnp.int32),
        pltpu.VMEM((IPW,), jnp.int32),
        pltpu.VMEM((NTILES, 8, 128), jnp.float32),
        pltpu.VMEM((NTILES, 8, 128), jnp.float32),
        pltpu.SemaphoreType.DMA,
        pltpu.SemaphoreType.DMA,
        pltpu.SemaphoreType.DMA,
        pltpu.SemaphoreType.DMA,
    ],
)
def _sc_embed(wt_hbm, idxt_hbm, out_hbm, table_v, idx0, idx1, st0, st1,
              isem0, isem1, osem0, osem1):
    wid = lax.axis_index("s") * NC + lax.axis_index("c")
    i0 = wid * IPW

    pltpu.sync_copy(wt_hbm, table_v)

    idx_bufs = (idx0, idx1)
    st_bufs = (st0, st1)
    isems = (isem0, isem1)
    osems = (osem0, osem1)

    for b in range(2):
        pltpu.async_copy(
            idxt_hbm.at[pl.ds(b * NI + i0, IPW)], idx_bufs[b], isems[b]
        )

    def compute_chunk(idx_st, st):
        @plsc.parallel_loop(0, NSEG, unroll=2)
        def seg(s):
            ib = lax.div(s, NKB)
            v = lax.rem(s, NKB)
            rowvec = idx_st[pl.ds(s * L, L)]
            for kp in range(D):
                vals = plsc.load_gather(table_v, [rowvec + kp * V])
                st[(kp // 8) * NIB + ib, kp % 8, pl.ds(v * L, L)] = vals

    def fire_out(st, j, sem):
        @plsc.parallel_loop(0, NTILES)
        def fire(t):
            kb = lax.div(t, NIB)
            ib = lax.rem(t, NIB)
            pltpu.async_copy(
                st.at[t],
                out_hbm.at[j, pl.ds(kb * 8, 8), pl.ds(i0 + ib * 128, 128)],
                sem,
            )

    def drain_out(st, sem):
        @plsc.parallel_loop(0, NTILES)
        def drain(t):
            pltpu.make_async_copy(
                st.at[0], out_hbm.at[0, pl.ds(0, 8), pl.ds(0, 128)], sem
            ).wait()

    def super_step(ss, carry):
        for b in range(2):
            j = 2 * ss + b
            pltpu.make_async_copy(
                idxt_hbm.at[pl.ds(0, IPW)], idx_bufs[b], isems[b]
            ).wait()

            @pl.when(ss > 0)
            def _drain():
                drain_out(st_bufs[b], osems[b])

            compute_chunk(idx_bufs[b], st_bufs[b])
            fire_out(st_bufs[b], j, osems[b])

            @pl.when(j + 2 < NJ)
            def _prefetch():
                pltpu.async_copy(
                    idxt_hbm.at[pl.ds((j + 2) * NI + i0, IPW)],
                    idx_bufs[b],
                    isems[b],
                )
        return carry

    lax.fori_loop(0, NJ // 2, super_step, 0)

    for b in range(2):
        drain_out(st_bufs[b], osems[b])


def kernel(channel_indices, weight):
    idxt = channel_indices.T.reshape(-1).astype(jnp.int32)
    wt = weight.T.reshape(-1)
    out = _sc_embed(wt, idxt)
    return jnp.transpose(out, (2, 0, 1))


# speedup vs baseline: 47.2091x; 1.5144x over previous
import functools

import jax
import jax.numpy as jnp
from jax import lax
from jax.experimental import pallas as pl
from jax.experimental.pallas import tpu as pltpu
from jax.experimental.pallas import tpu_sc as plsc

NI, NJ = 16384, 100
V, D = 256, 64
NC, NS, L = 2, 16, 16
NW = NC * NS
IPW = NI // NW
NIB = IPW // 128
NKB = D // 8
NTILES = NKB * NIB
NSEG = IPW // L

_mesh = plsc.VectorSubcoreMesh(
    core_axis_name="c", subcore_axis_name="s", num_cores=NC, num_subcores=NS
)


@functools.partial(
    pl.kernel,
    out_type=jax.ShapeDtypeStruct((NJ, D, NI), jnp.float32),
    mesh=_mesh,
    compiler_params=pltpu.CompilerParams(needs_layout_passes=False),
    scratch_types=[
        pltpu.VMEM((D * V,), jnp.float32),
        pltpu.VMEM((IPW,), jnp.int32),
        pltpu.VMEM((IPW,), jnp.int32),
        pltpu.VMEM((NTILES, 8, 128), jnp.float32),
        pltpu.VMEM((NTILES, 8, 128), jnp.float32),
        pltpu.SemaphoreType.DMA,
        pltpu.SemaphoreType.DMA,
        pltpu.SemaphoreType.DMA,
        pltpu.SemaphoreType.DMA,
    ],
)
def _sc_embed(wt_hbm, idxt_hbm, out_hbm, table_v, idx0, idx1, st0, st1,
              isem0, isem1, osem0, osem1):
    wid = lax.axis_index("s") * NC + lax.axis_index("c")
    i0 = wid * IPW

    pltpu.sync_copy(wt_hbm, table_v)

    idx_bufs = (idx0, idx1)
    st_bufs = (st0, st1)
    isems = (isem0, isem1)
    osems = (osem0, osem1)

    for b in range(2):
        pltpu.async_copy(
            idxt_hbm.at[pl.ds(b * NI + i0, IPW)], idx_bufs[b], isems[b]
        )

    def compute_chunk(idx_st, st):
        @plsc.parallel_loop(0, NSEG, unroll=4)
        def seg(s):
            ib = lax.div(s, NKB)
            v = lax.rem(s, NKB)
            rowvec = idx_st[pl.ds(s * L, L)]
            for kp in range(D):
                vals = plsc.load_gather(table_v, [rowvec + kp * V])
                st[(kp // 8) * NIB + ib, kp % 8, pl.ds(v * L, L)] = vals

    def fire_out(st, j, sem):
        @plsc.parallel_loop(0, NTILES)
        def fire(t):
            kb = lax.div(t, NIB)
            ib = lax.rem(t, NIB)
            pltpu.async_copy(
                st.at[t],
                out_hbm.at[j, pl.ds(kb * 8, 8), pl.ds(i0 + ib * 128, 128)],
                sem,
            )

    def drain_out(st, sem):
        @plsc.parallel_loop(0, NTILES)
        def drain(t):
            pltpu.make_async_copy(
                st.at[0], out_hbm.at[0, pl.ds(0, 8), pl.ds(0, 128)], sem
            ).wait()

    def super_step(ss, carry):
        for b in range(2):
            j = 2 * ss + b
            pltpu.make_async_copy(
                idxt_hbm.at[pl.ds(0, IPW)], idx_bufs[b], isems[b]
            ).wait()

            @pl.when(ss > 0)
            def _drain():
                drain_out(st_bufs[b], osems[b])

            compute_chunk(idx_bufs[b], st_bufs[b])
            fire_out(st_bufs[b], j, osems[b])

            @pl.when(j + 2 < NJ)
            def _prefetch():
                pltpu.async_copy(
                    idxt_hbm.at[pl.ds((j + 2) * NI + i0, IPW)],
                    idx_bufs[b],
                    isems[b],
                )
        return carry

    lax.fori_loop(0, NJ // 2, super_step, 0)

    for b in range(2):
        drain_out(st_bufs[b], osems[b])


def kernel(channel_indices, weight):
    idxt = channel_indices.T.reshape(-1).astype(jnp.int32)
    wt = weight.T.reshape(-1)
    out = _sc_embed(wt, idxt)
    return jnp.transpose(out, (2, 0, 1))
